# baseline (speedup 1.0000x reference)
import jax
import jax.numpy as jnp
from jax import lax
from jax.experimental import pallas as pl
from jax.experimental.pallas import tpu as pltpu
from jax.experimental.pallas import tpu_sc as plsc

B = 16384
NODES = 1000000
K = 64
KP = 128
H = 128
LH = 384

NC = 2
NS = 16
NW = NC * NS
CHUNK = 2 * B // NW
IDXW = 128
NJ = CHUNK // IDXW
HALF = CHUNK // 2
NJH = NJ // 2

BLK1 = 4096
BLK2 = 4096


def _cluster_body(h_ref, t_ref, wt_ref, znt_ref, b_ref, o_ref):
    wt = wt_ref[...]
    znt = znt_ref[...]
    bb = b_ref[...]
    for side, x_ref in ((0, h_ref), (1, t_ref)):
        xo = jnp.dot(x_ref[...], wt, preferred_element_type=jnp.float32) + bb
        lg = jnp.dot(xo, znt, preferred_element_type=jnp.float32)
        m = jnp.max(lg, axis=-1, keepdims=True)
        e = jnp.exp(lg - m)
        probs = e / jnp.sum(e, axis=-1, keepdims=True)
        o_ref[side] = jnp.concatenate(
            [probs, jnp.zeros((probs.shape[0], KP - K), jnp.float32)], axis=1)


def _cluster_call(head_embs, tail_embs, wt, znt, b2):
    grid = (B // BLK1,)
    return pl.pallas_call(
        _cluster_body,
        grid=grid,
        in_specs=[
            pl.BlockSpec((BLK1, LH), lambda i: (i, 0)),
            pl.BlockSpec((BLK1, LH), lambda i: (i, 0)),
            pl.BlockSpec((LH, H), lambda i: (0, 0)),
            pl.BlockSpec((H, K), lambda i: (0, 0)),
            pl.BlockSpec((1, H), lambda i: (0, 0)),
        ],
        out_specs=pl.BlockSpec((2, BLK1, KP), lambda i: (0, i, 0)),
        out_shape=jax.ShapeDtypeStruct((2, B, KP), jnp.float32),
    )(head_embs, tail_embs, wt, znt, b2)


QTR = CHUNK // 4
NJQ = NJ // 4


def _sg_body(cl, idx3, val3, g, tab, idx_v, val_v, rows_a, rows_b, gsem, wsem):
    c = lax.axis_index("c")
    s = lax.axis_index("s")
    wid = c * NS + s
    base = wid * CHUNK
    pltpu.sync_copy(idx3.at[wid], idx_v)
    pltpu.sync_copy(val3.at[wid], val_v)
    for j in range(NJ):
        pltpu.sync_copy(val_v.at[j], tab.at[idx_v.at[j]])
    plsc.subcore_barrier()
    for j in range(NJ):
        pltpu.sync_copy(tab.at[idx_v.at[j]], val_v.at[j])
    bufs = (rows_a, rows_b)

    def _gathers(q):
        return [
            pltpu.async_copy(
                cl.at[val_v.at[q * NJQ + j]],
                bufs[q % 2].at[pl.ds(j * IDXW, IDXW)], gsem)
            for j in range(NJQ)
        ]

    gcps = {0: _gathers(0)}
    wcps = {}
    for q in range(4):
        for cp in gcps[q]:
            cp.wait()
        wcps[q] = pltpu.async_copy(
            bufs[q % 2], g.at[pl.ds(base + q * QTR, QTR)], wsem)
        if q + 1 < 4:
            if q >= 1:
                wcps[q - 1].wait()
            gcps[q + 1] = _gathers(q + 1)
    wcps[2].wait()
    wcps[3].wait()


def _sg_call(cl, idx3, val3):
    f = pl.kernel(
        _sg_body,
        out_type=jax.ShapeDtypeStruct((2 * B, KP), jnp.float32),
        mesh=plsc.VectorSubcoreMesh(
            core_axis_name="c", subcore_axis_name="s", num_cores=NC,
            num_subcores=NS),
        scratch_types=[
            pltpu.VMEM_SHARED((NODES,), jnp.int32),
            pltpu.VMEM((NJ, IDXW), jnp.int32),
            pltpu.VMEM((NJ, IDXW), jnp.int32),
            pltpu.VMEM((QTR, KP), jnp.float32),
            pltpu.VMEM((QTR, KP), jnp.float32),
            pltpu.SemaphoreType.DMA,
            pltpu.SemaphoreType.DMA,
        ],
    )
    return f(cl, idx3, val3)


def _final_body(gh_ref, gt_ref, zn_ref, o_ref):
    zn = zn_ref[...]
    s1 = jax.nn.sigmoid(
        jnp.dot(gh_ref[:, :K], zn, preferred_element_type=jnp.float32))
    s2 = jax.nn.sigmoid(
        jnp.dot(gt_ref[:, :K], zn, preferred_element_type=jnp.float32))
    o_ref[...] = s1 * s2


def _final_call(g, Zn):
    nblk = B // BLK2
    return pl.pallas_call(
        _final_body,
        grid=(nblk,),
        in_specs=[
            pl.BlockSpec((BLK2, KP), lambda i: (i, 0)),
            pl.BlockSpec((BLK2, KP), lambda i: (i + B // BLK2, 0)),
            pl.BlockSpec((K, H), lambda i: (0, 0)),
        ],
        out_specs=pl.BlockSpec((BLK2, H), lambda i: (i, 0)),
        out_shape=jax.ShapeDtypeStruct((B, H), jnp.float32),
    )(g, g, Zn)


def kernel(head_embs, tail_embs, Hn, Zn, W, b, head_idx, tail_idx):
    del Hn
    wt = W.T
    znt = Zn.T
    b2 = b.reshape(1, H)
    cl2 = _cluster_call(head_embs, tail_embs, wt, znt, b2)
    cl = cl2.reshape(2 * B, KP)
    idx3 = jnp.concatenate([head_idx, tail_idx]).reshape(NW, NJ, IDXW)
    val3 = jnp.arange(2 * B, dtype=jnp.int32).reshape(NW, NJ, IDXW)
    g = _sg_call(cl, idx3, val3)
    return _final_call(g, Zn)


# speedup vs baseline: 1.0637x; 1.0637x over previous
import jax
import jax.numpy as jnp
from jax import lax
from jax.experimental import pallas as pl
from jax.experimental.pallas import tpu as pltpu
from jax.experimental.pallas import tpu_sc as plsc

B = 16384
NODES = 1000000
K = 64
KP = 128
H = 128
LH = 384

NC = 2
NS = 16
NW = NC * NS
CHUNK = 2 * B // NW
IDXW = 128
NJ = CHUNK // IDXW
HALF = CHUNK // 2
NJH = NJ // 2

BLK1 = 4096
BLK2 = 4096


def _cluster_body(h_ref, t_ref, wt_ref, znt_ref, b_ref, o_ref):
    wt = wt_ref[...]
    znt = znt_ref[...]
    bb = b_ref[...]
    for side, x_ref in ((0, h_ref), (1, t_ref)):
        xo = jnp.dot(x_ref[...], wt, preferred_element_type=jnp.float32) + bb
        lg = jnp.dot(xo, znt, preferred_element_type=jnp.float32)
        m = jnp.max(lg, axis=-1, keepdims=True)
        e = jnp.exp(lg - m)
        probs = e / jnp.sum(e, axis=-1, keepdims=True)
        o_ref[side] = jnp.concatenate(
            [probs, jnp.zeros((probs.shape[0], KP - K), jnp.float32)], axis=1)


def _cluster_call(head_embs, tail_embs, wt, znt, b2):
    grid = (B // BLK1,)
    return pl.pallas_call(
        _cluster_body,
        grid=grid,
        in_specs=[
            pl.BlockSpec((BLK1, LH), lambda i: (i, 0)),
            pl.BlockSpec((BLK1, LH), lambda i: (i, 0)),
            pl.BlockSpec((LH, H), lambda i: (0, 0)),
            pl.BlockSpec((H, K), lambda i: (0, 0)),
            pl.BlockSpec((1, H), lambda i: (0, 0)),
        ],
        out_specs=pl.BlockSpec((2, BLK1, KP), lambda i: (0, i, 0)),
        out_shape=jax.ShapeDtypeStruct((2, B, KP), jnp.float32),
    )(head_embs, tail_embs, wt, znt, b2)


def _dedup_body(idx3, val3, w3, tab, idx_v, val_v):
    c = lax.axis_index("c")
    s = lax.axis_index("s")
    wid = c * NS + s
    pltpu.sync_copy(idx3.at[wid], idx_v)
    pltpu.sync_copy(val3.at[wid], val_v)
    for j in range(NJ):
        pltpu.sync_copy(val_v.at[j], tab.at[idx_v.at[j]])
    plsc.subcore_barrier()
    for j in range(NJ):
        pltpu.sync_copy(tab.at[idx_v.at[j]], val_v.at[j])
    pltpu.sync_copy(val_v, w3.at[wid])


def _dedup_call(idx3, val3):
    f = pl.kernel(
        _dedup_body,
        out_type=jax.ShapeDtypeStruct((NW, NJ, IDXW), jnp.int32),
        mesh=plsc.VectorSubcoreMesh(
            core_axis_name="c", subcore_axis_name="s", num_cores=NC,
            num_subcores=NS),
        scratch_types=[
            pltpu.VMEM_SHARED((NODES,), jnp.int32),
            pltpu.VMEM((NJ, IDXW), jnp.int32),
            pltpu.VMEM((NJ, IDXW), jnp.int32),
        ],
    )
    return f(idx3, val3)


def _rows_body(cl, w3, g, val_v, rows_v, sem):
    c = lax.axis_index("c")
    s = lax.axis_index("s")
    wid = c * NS + s
    base = wid * CHUNK
    pltpu.sync_copy(w3.at[wid], val_v)
    for h in range(2):
        cps = [
            pltpu.async_copy(
                cl.at[val_v.at[h * NJH + j]],
                rows_v.at[pl.ds(j * IDXW, IDXW)], sem)
            for j in range(NJH)
        ]
        for cp in cps:
            cp.wait()
        pltpu.sync_copy(rows_v, g.at[pl.ds(base + h * HALF, HALF)])


def _rows_call(cl, w3):
    f = pl.kernel(
        _rows_body,
        out_type=jax.ShapeDtypeStruct((2 * B, KP), jnp.float32),
        mesh=plsc.VectorSubcoreMesh(
            core_axis_name="c", subcore_axis_name="s", num_cores=NC,
            num_subcores=NS),
        scratch_types=[
            pltpu.VMEM((NJ, IDXW), jnp.int32),
            pltpu.VMEM((HALF, KP), jnp.float32),
            pltpu.SemaphoreType.DMA,
        ],
    )
    return f(cl, w3)


def _final_body(gh_ref, gt_ref, zn_ref, o_ref):
    zn = zn_ref[...]
    s1 = jax.nn.sigmoid(
        jnp.dot(gh_ref[:, :K], zn, preferred_element_type=jnp.float32))
    s2 = jax.nn.sigmoid(
        jnp.dot(gt_ref[:, :K], zn, preferred_element_type=jnp.float32))
    o_ref[...] = s1 * s2


def _final_call(g, Zn):
    nblk = B // BLK2
    return pl.pallas_call(
        _final_body,
        grid=(nblk,),
        in_specs=[
            pl.BlockSpec((BLK2, KP), lambda i: (i, 0)),
            pl.BlockSpec((BLK2, KP), lambda i: (i + B // BLK2, 0)),
            pl.BlockSpec((K, H), lambda i: (0, 0)),
        ],
        out_specs=pl.BlockSpec((BLK2, H), lambda i: (i, 0)),
        out_shape=jax.ShapeDtypeStruct((B, H), jnp.float32),
    )(g, g, Zn)


def kernel(head_embs, tail_embs, Hn, Zn, W, b, head_idx, tail_idx):
    del Hn
    wt = W.T
    znt = Zn.T
    b2 = b.reshape(1, H)
    cl2 = _cluster_call(head_embs, tail_embs, wt, znt, b2)
    cl = cl2.reshape(2 * B, KP)
    idx3 = jnp.concatenate([head_idx, tail_idx]).reshape(NW, NJ, IDXW)
    val3 = jnp.arange(2 * B, dtype=jnp.int32).reshape(NW, NJ, IDXW)
    w3 = _dedup_call(idx3, val3)
    g = _rows_call(cl, w3)
    return _final_call(g, Zn)
